# Initial kernel scaffold; baseline (speedup 1.0000x reference)
import jax
import jax.numpy as jnp
from jax.experimental import pallas as pl


def kernel(x, edge_index, batch, W1, b1, W2, b2, W3, b3, We, be):
    raise NotImplementedError("write your pallas kernel here")


# speedup vs baseline: 6.0156x; 6.0156x over previous
import functools

import jax
import jax.numpy as jnp
from jax import lax
from jax.experimental import pallas as pl
from jax.experimental.pallas import tpu as pltpu
from jax.experimental.pallas import tpu_sc as plsc

N_NODES = 10000
N_EDGES = 320000
IN_DIM = 128
HID_DIM = 256
OUT_DIM = 128
NUM_GRAPHS = 64

NC = 2
NS = 16

E_PER_TILE = 10240
E_PAD = E_PER_TILE * NC * NS
CHUNK = 128
N_CHUNKS = E_PER_TILE // CHUNK
N_ACC = 10240
ROWS_PER_TILE = N_ACC // NS
FH = 64

ROW_TILE = 1000
GRID = N_NODES // ROW_TILE

_HIGH = lax.Precision.HIGHEST


def _sc_degree(dst3, e016, zero16):
    mesh = plsc.VectorSubcoreMesh(core_axis_name="c", subcore_axis_name="s")

    @functools.partial(
        pl.kernel,
        mesh=mesh,
        compiler_params=pltpu.CompilerParams(use_tc_tiling_on_sc=False),
        out_type=jax.ShapeDtypeStruct((NC, N_ACC, 16), jnp.float32),
        scratch_types=[
            pltpu.VMEM((N_CHUNKS, CHUNK), jnp.int32),
            pltpu.VMEM((CHUNK, 16), jnp.float32),
            pltpu.VMEM_SHARED((N_ACC, 16), jnp.float32),
            pltpu.SemaphoreType.DMA,
        ],
    )
    def body(dst_hbm, e0_hbm, z_hbm, out_hbm, dst_v, val, acc, sem):
        c = lax.axis_index("c")
        s = lax.axis_index("s")
        t = c * NS + s

        base = s * ROWS_PER_TILE
        pltpu.sync_copy(z_hbm, val)

        @pl.loop(0, ROWS_PER_TILE // CHUNK)
        def _(j):
            pltpu.sync_copy(val, acc.at[pl.ds(base + j * CHUNK, CHUNK)])

        pltpu.sync_copy(e0_hbm, val)
        pltpu.sync_copy(dst_hbm.at[pl.ds(t * N_CHUNKS, N_CHUNKS)], dst_v)
        plsc.subcore_barrier()

        @pl.loop(0, N_CHUNKS)
        def _(i):
            pltpu.async_copy(val, acc.at[dst_v.at[i]], sem, add=True).wait()

        plsc.subcore_barrier()

        @pl.loop(0, ROWS_PER_TILE // CHUNK)
        def _(j):
            off = base + j * CHUNK
            pltpu.sync_copy(acc.at[pl.ds(off, CHUNK)], val)
            pltpu.sync_copy(val, out_hbm.at[c, pl.ds(off, CHUNK)])

    return body(dst3, e016, zero16)


def _sc_edge_pass(g, src2, dst3, zeroF):
    mesh = plsc.VectorSubcoreMesh(core_axis_name="c", subcore_axis_name="s")

    @functools.partial(
        pl.kernel,
        mesh=mesh,
        compiler_params=pltpu.CompilerParams(use_tc_tiling_on_sc=False),
        out_type=jax.ShapeDtypeStruct((NC, N_ACC, FH), jnp.float32),
        scratch_types=[
            pltpu.VMEM((N_CHUNKS, CHUNK), jnp.int32),
            pltpu.VMEM((N_CHUNKS, CHUNK), jnp.int32),
            pltpu.VMEM((CHUNK, FH), jnp.float32),
            pltpu.VMEM((CHUNK, FH), jnp.float32),
            pltpu.VMEM_SHARED((N_ACC, FH), jnp.float32),
            pltpu.SemaphoreType.DMA,
            pltpu.SemaphoreType.DMA,
            pltpu.SemaphoreType.DMA,
        ],
    )
    def body(g_hbm, src_hbm, dst_hbm, z_hbm, out_hbm,
             src_v, dst_v, rows0, rows1, acc, sem_g0, sem_g1, sem_s):
        c = lax.axis_index("c")
        s = lax.axis_index("s")
        t = c * NS + s

        pltpu.sync_copy(z_hbm, rows0)
        base = s * ROWS_PER_TILE

        @pl.loop(0, ROWS_PER_TILE // CHUNK)
        def _(j):
            pltpu.sync_copy(rows0, acc.at[pl.ds(base + j * CHUNK, CHUNK)])

        pltpu.sync_copy(src_hbm.at[pl.ds(t * N_CHUNKS, N_CHUNKS)], src_v)
        pltpu.sync_copy(dst_hbm.at[pl.ds(t * N_CHUNKS, N_CHUNKS)], dst_v)
        plsc.subcore_barrier()

        pltpu.async_copy(g_hbm.at[src_v.at[0]], rows0, sem_g0).wait()

        @pl.loop(0, N_CHUNKS - 1)
        def _(i):
            b = i % 2

            @pl.when(b == 0)
            def _():
                gat = pltpu.async_copy(g_hbm.at[src_v.at[i + 1]], rows1, sem_g1)
                pltpu.async_copy(rows0, acc.at[dst_v.at[i]], sem_s,
                                 add=True).wait()
                gat.wait()

            @pl.when(b == 1)
            def _():
                gat = pltpu.async_copy(g_hbm.at[src_v.at[i + 1]], rows0, sem_g0)
                pltpu.async_copy(rows1, acc.at[dst_v.at[i]], sem_s,
                                 add=True).wait()
                gat.wait()

        last = N_CHUNKS - 1

        @pl.when(last % 2 == 0)
        def _():
            pltpu.async_copy(rows0, acc.at[dst_v.at[last]], sem_s,
                             add=True).wait()

        @pl.when(last % 2 == 1)
        def _():
            pltpu.async_copy(rows1, acc.at[dst_v.at[last]], sem_s,
                             add=True).wait()

        plsc.subcore_barrier()

        @pl.loop(0, ROWS_PER_TILE // CHUNK)
        def _(j):
            off = base + j * CHUNK
            pltpu.sync_copy(acc.at[pl.ds(off, CHUNK)], rows0)
            pltpu.sync_copy(rows0, out_hbm.at[c, pl.ds(off, CHUNK)])

    return body(g, src2, dst3, zeroF)


def _dot(a, b):
    return lax.dot_general(a, b, (((1,), (0,)), ((), ())),
                           precision=_HIGH, preferred_element_type=jnp.float32)


def _acc_spec():
    return pl.BlockSpec((NC, ROW_TILE, FH), lambda i: (0, i, 0))


def _half_spec():
    return pl.BlockSpec((ROW_TILE, FH), lambda i: (i, 0))


def _comb(p_ref, g_ref):
    return p_ref[0] + p_ref[1] + g_ref[...]


def _k_prep(degp, x):
    def body(degp_ref, x_ref, dinv_ref, g0a_ref, g0b_ref):
        deg = degp_ref[0, :, 0:1] + degp_ref[1, :, 0:1] + 1.0
        dinv = lax.rsqrt(deg)
        dinv_ref[...] = dinv
        g0 = dinv * x_ref[...]
        g0a_ref[...] = g0[:, :FH]
        g0b_ref[...] = g0[:, FH:]

    return pl.pallas_call(
        body,
        grid=(GRID,),
        in_specs=[
            pl.BlockSpec((NC, ROW_TILE, 16), lambda i: (0, i, 0)),
            pl.BlockSpec((ROW_TILE, IN_DIM), lambda i: (i, 0)),
        ],
        out_specs=[
            pl.BlockSpec((ROW_TILE, 1), lambda i: (i, 0)),
            _half_spec(),
            _half_spec(),
        ],
        out_shape=[
            jax.ShapeDtypeStruct((N_NODES, 1), jnp.float32),
            jax.ShapeDtypeStruct((N_NODES, FH), jnp.float32),
            jax.ShapeDtypeStruct((N_NODES, FH), jnp.float32),
        ],
    )(degp, x)


def _k_layer1(p0, p1, g0a, g0b, dinv, W1, b1):
    def body(p0_ref, p1_ref, g0a_ref, g0b_ref, dinv_ref, W1_ref, b1_ref,
             ga_ref, gb_ref, gc_ref, gd_ref):
        dinv = dinv_ref[...]
        z = dinv * jnp.concatenate(
            [_comb(p0_ref, g0a_ref), _comb(p1_ref, g0b_ref)], axis=1)
        h = jax.nn.relu(_dot(z, W1_ref[...]) + b1_ref[...])
        ga_ref[...] = dinv * h[:, :FH]
        gb_ref[...] = dinv * h[:, FH:2 * FH]
        gc_ref[...] = dinv * h[:, 2 * FH:3 * FH]
        gd_ref[...] = dinv * h[:, 3 * FH:]

    return pl.pallas_call(
        body,
        grid=(GRID,),
        in_specs=[
            _acc_spec(),
            _acc_spec(),
            _half_spec(),
            _half_spec(),
            pl.BlockSpec((ROW_TILE, 1), lambda i: (i, 0)),
            pl.BlockSpec((IN_DIM, HID_DIM), lambda i: (0, 0)),
            pl.BlockSpec((1, HID_DIM), lambda i: (0, 0)),
        ],
        out_specs=[_half_spec() for _ in range(4)],
        out_shape=[jax.ShapeDtypeStruct((N_NODES, FH), jnp.float32)
                   for _ in range(4)],
    )(p0, p1, g0a, g0b, dinv, W1, b1)


def _k_layer23(q0, q1, q2, q3, g1a, g1b, g1c, g1d, dinv, W2, b2, W3):
    def body(q0_ref, q1_ref, q2_ref, q3_ref, ga_ref, gb_ref, gc_ref, gd_ref,
             dinv_ref, W2_ref, b2_ref, W3_ref, g2a_ref, g2b_ref):
        dinv = dinv_ref[...]
        z = dinv * jnp.concatenate(
            [_comb(q0_ref, ga_ref), _comb(q1_ref, gb_ref),
             _comb(q2_ref, gc_ref), _comb(q3_ref, gd_ref)], axis=1)
        h2 = jax.nn.relu(_dot(z, W2_ref[...]) + b2_ref[...])
        m = _dot(h2, W3_ref[...])
        g2 = dinv * m
        g2a_ref[...] = g2[:, :FH]
        g2b_ref[...] = g2[:, FH:]

    return pl.pallas_call(
        body,
        grid=(GRID,),
        in_specs=[
            _acc_spec(), _acc_spec(), _acc_spec(), _acc_spec(),
            _half_spec(), _half_spec(), _half_spec(), _half_spec(),
            pl.BlockSpec((ROW_TILE, 1), lambda i: (i, 0)),
            pl.BlockSpec((HID_DIM, HID_DIM), lambda i: (0, 0)),
            pl.BlockSpec((1, HID_DIM), lambda i: (0, 0)),
            pl.BlockSpec((HID_DIM, OUT_DIM), lambda i: (0, 0)),
        ],
        out_specs=[_half_spec(), _half_spec()],
        out_shape=[jax.ShapeDtypeStruct((N_NODES, FH), jnp.float32)
                   for _ in range(2)],
    )(q0, q1, q2, q3, g1a, g1b, g1c, g1d, dinv, W2, b2, W3)


def _k_final(r0, r1, g2a, g2b, dinv, b3, batch2, We, be):
    def body(r0_ref, r1_ref, g2a_ref, g2b_ref, dinv_ref, b3_ref, batch_ref,
             We_ref, be_ref, h_ref, emb_ref, pool_acc, cnt_acc):
        i = pl.program_id(0)
        h3 = dinv_ref[...] * jnp.concatenate(
            [_comb(r0_ref, g2a_ref), _comb(r1_ref, g2b_ref)], axis=1)
        h3 = h3 + b3_ref[...]
        h_ref[...] = h3

        seg = lax.broadcasted_iota(jnp.int32, (ROW_TILE, NUM_GRAPHS), 1)
        oh = (batch_ref[...] == seg).astype(jnp.float32)

        @pl.when(i == 0)
        def _():
            pool_acc[...] = jnp.zeros_like(pool_acc)
            cnt_acc[...] = jnp.zeros_like(cnt_acc)

        contract = (((0,), (0,)), ((), ()))
        pool_acc[...] += lax.dot_general(
            oh, h3, contract, precision=_HIGH,
            preferred_element_type=jnp.float32)
        cnt_acc[...] += lax.dot_general(
            oh, jnp.ones((ROW_TILE, OUT_DIM), jnp.float32), contract,
            precision=_HIGH, preferred_element_type=jnp.float32)

        @pl.when(i == GRID - 1)
        def _():
            mean = pool_acc[...] / jnp.maximum(cnt_acc[...], 1.0)
            emb_ref[...] = _dot(mean, We_ref[...]) + be_ref[...]

    return pl.pallas_call(
        body,
        grid=(GRID,),
        in_specs=[
            _acc_spec(), _acc_spec(),
            _half_spec(), _half_spec(),
            pl.BlockSpec((ROW_TILE, 1), lambda i: (i, 0)),
            pl.BlockSpec((1, OUT_DIM), lambda i: (0, 0)),
            pl.BlockSpec((ROW_TILE, 1), lambda i: (i, 0)),
            pl.BlockSpec((OUT_DIM, OUT_DIM), lambda i: (0, 0)),
            pl.BlockSpec((1, OUT_DIM), lambda i: (0, 0)),
        ],
        out_specs=[
            pl.BlockSpec((ROW_TILE, OUT_DIM), lambda i: (i, 0)),
            pl.BlockSpec((NUM_GRAPHS, OUT_DIM), lambda i: (0, 0)),
        ],
        out_shape=[
            jax.ShapeDtypeStruct((N_NODES, OUT_DIM), jnp.float32),
            jax.ShapeDtypeStruct((NUM_GRAPHS, OUT_DIM), jnp.float32),
        ],
        scratch_shapes=[
            pltpu.VMEM((NUM_GRAPHS, OUT_DIM), jnp.float32),
            pltpu.VMEM((NUM_GRAPHS, OUT_DIM), jnp.float32),
        ],
    )(r0, r1, g2a, g2b, dinv, b3, batch2, We, be)


def kernel(x, edge_index, batch, W1, b1, W2, b2, W3, b3, We, be):
    src = edge_index[0].astype(jnp.int32)
    dst = edge_index[1].astype(jnp.int32)
    npad = E_PAD - N_EDGES
    src2 = jnp.concatenate([src, jnp.zeros((npad,), jnp.int32)])
    src2 = src2.reshape(E_PAD // CHUNK, CHUNK)
    dst3 = jnp.concatenate(
        [dst, jnp.full((npad,), N_NODES, jnp.int32)])
    dst3 = dst3.reshape(E_PAD // CHUNK, CHUNK)
    batch2 = batch.astype(jnp.int32).reshape(N_NODES, 1)
    b1r = b1.reshape(1, HID_DIM)
    b2r = b2.reshape(1, HID_DIM)
    b3r = b3.reshape(1, OUT_DIM)
    ber = be.reshape(1, OUT_DIM)
    e016 = jnp.tile((jnp.arange(16) == 0).astype(jnp.float32), (CHUNK, 1))
    zero16 = jnp.zeros((CHUNK, 16), jnp.float32)
    zeroF = jnp.zeros((CHUNK, FH), jnp.float32)

    degp = _sc_degree(dst3, e016, zero16)
    dinv, g0a, g0b = _k_prep(degp, x)
    p0 = _sc_edge_pass(g0a, src2, dst3, zeroF)
    p1 = _sc_edge_pass(g0b, src2, dst3, zeroF)
    g1a, g1b, g1c, g1d = _k_layer1(p0, p1, g0a, g0b, dinv, W1, b1r)
    q0 = _sc_edge_pass(g1a, src2, dst3, zeroF)
    q1 = _sc_edge_pass(g1b, src2, dst3, zeroF)
    q2 = _sc_edge_pass(g1c, src2, dst3, zeroF)
    q3 = _sc_edge_pass(g1d, src2, dst3, zeroF)
    g2a, g2b = _k_layer23(q0, q1, q2, q3, g1a, g1b, g1c, g1d,
                          dinv, W2, b2r, W3)
    r0 = _sc_edge_pass(g2a, src2, dst3, zeroF)
    r1 = _sc_edge_pass(g2b, src2, dst3, zeroF)
    h, emb = _k_final(r0, r1, g2a, g2b, dinv, b3r, batch2, We, ber)
    return (emb, h)


# speedup vs baseline: 6.5791x; 1.0937x over previous
import functools

import jax
import jax.numpy as jnp
from jax import lax
from jax.experimental import pallas as pl
from jax.experimental.pallas import tpu as pltpu
from jax.experimental.pallas import tpu_sc as plsc

N_NODES = 10000
N_EDGES = 320000
IN_DIM = 128
HID_DIM = 256
OUT_DIM = 128
NUM_GRAPHS = 64

NC = 2
NS = 16

E_PER_TILE = 10240
E_PAD = E_PER_TILE * NC * NS
CHUNK = 128
N_CHUNKS = E_PER_TILE // CHUNK
N_ACC = 10240
ROWS_PER_TILE = N_ACC // NS
FH = 64
NBUF = 4
AHEAD = 2

ROW_TILE = 1000
GRID = N_NODES // ROW_TILE

_HIGH = lax.Precision.HIGHEST


def _sc_degree(dst3, e016, zero16):
    mesh = plsc.VectorSubcoreMesh(core_axis_name="c", subcore_axis_name="s")

    @functools.partial(
        pl.kernel,
        mesh=mesh,
        compiler_params=pltpu.CompilerParams(use_tc_tiling_on_sc=False),
        out_type=jax.ShapeDtypeStruct((NC, N_ACC, 16), jnp.float32),
        scratch_types=[
            pltpu.VMEM((N_CHUNKS, CHUNK), jnp.int32),
            pltpu.VMEM((CHUNK, 16), jnp.float32),
            pltpu.VMEM_SHARED((N_ACC, 16), jnp.float32),
            pltpu.SemaphoreType.DMA,
        ],
    )
    def body(dst_hbm, e0_hbm, z_hbm, out_hbm, dst_v, val, acc, sem):
        c = lax.axis_index("c")
        s = lax.axis_index("s")
        t = c * NS + s

        base = s * ROWS_PER_TILE
        pltpu.sync_copy(z_hbm, val)

        @pl.loop(0, ROWS_PER_TILE // CHUNK)
        def _(j):
            pltpu.sync_copy(val, acc.at[pl.ds(base + j * CHUNK, CHUNK)])

        pltpu.sync_copy(e0_hbm, val)
        pltpu.sync_copy(dst_hbm.at[pl.ds(t * N_CHUNKS, N_CHUNKS)], dst_v)
        plsc.subcore_barrier()

        @pl.loop(0, N_CHUNKS)
        def _(i):
            pltpu.async_copy(val, acc.at[dst_v.at[i]], sem, add=True).wait()

        plsc.subcore_barrier()

        @pl.loop(0, ROWS_PER_TILE // CHUNK)
        def _(j):
            off = base + j * CHUNK
            pltpu.sync_copy(acc.at[pl.ds(off, CHUNK)], val)
            pltpu.sync_copy(val, out_hbm.at[c, pl.ds(off, CHUNK)])

    return body(dst3, e016, zero16)


def _sc_edge_pass(g, src2, dst3, zeroF):
    mesh = plsc.VectorSubcoreMesh(core_axis_name="c", subcore_axis_name="s")

    @functools.partial(
        pl.kernel,
        mesh=mesh,
        compiler_params=pltpu.CompilerParams(use_tc_tiling_on_sc=False),
        out_type=jax.ShapeDtypeStruct((NC, N_ACC, FH), jnp.float32),
        scratch_types=[
            pltpu.VMEM((N_CHUNKS, CHUNK), jnp.int32),
            pltpu.VMEM((N_CHUNKS, CHUNK), jnp.int32),
            [pltpu.VMEM((CHUNK, FH), jnp.float32) for _ in range(NBUF)],
            [pltpu.SemaphoreType.DMA for _ in range(NBUF)],
            [pltpu.SemaphoreType.DMA for _ in range(NBUF)],
            pltpu.VMEM_SHARED((N_ACC, FH), jnp.float32),
        ],
    )
    def body(g_hbm, src_hbm, dst_hbm, z_hbm, out_hbm,
             src_v, dst_v, rows, sem_g, sem_s, acc):
        c = lax.axis_index("c")
        s = lax.axis_index("s")
        t = c * NS + s

        pltpu.sync_copy(z_hbm, rows[0])
        base = s * ROWS_PER_TILE

        @pl.loop(0, ROWS_PER_TILE // CHUNK)
        def _(j):
            pltpu.sync_copy(rows[0], acc.at[pl.ds(base + j * CHUNK, CHUNK)])

        pltpu.sync_copy(src_hbm.at[pl.ds(t * N_CHUNKS, N_CHUNKS)], src_v)
        pltpu.sync_copy(dst_hbm.at[pl.ds(t * N_CHUNKS, N_CHUNKS)], dst_v)
        plsc.subcore_barrier()

        for b in range(AHEAD):
            pltpu.async_copy(g_hbm.at[src_v.at[b]], rows[b], sem_g[b])

        @pl.loop(0, N_CHUNKS // NBUF)
        def _(o):
            for b in range(NBUF):
                i = o * NBUF + b
                j_b = (b + AHEAD) % NBUF

                def prefetch(i=i, b=b, j_b=j_b):
                    j = i + AHEAD

                    def drain():
                        pltpu.make_async_copy(
                            rows[j_b], acc.at[dst_v.at[j - NBUF]],
                            sem_s[j_b]).wait()

                    if b + AHEAD >= NBUF:
                        drain()
                    else:
                        @pl.when(o >= 1)
                        def _():
                            drain()
                    pltpu.async_copy(g_hbm.at[src_v.at[j]], rows[j_b],
                                     sem_g[j_b])

                if b + AHEAD < NBUF:
                    prefetch()
                else:
                    @pl.when(o < N_CHUNKS // NBUF - 1)
                    def _():
                        prefetch()

                pltpu.make_async_copy(g_hbm.at[src_v.at[i]], rows[b],
                                      sem_g[b]).wait()
                pltpu.async_copy(rows[b], acc.at[dst_v.at[i]], sem_s[b],
                                 add=True)

        for k in range(NBUF):
            i = N_CHUNKS - NBUF + k
            pltpu.make_async_copy(rows[i % NBUF], acc.at[dst_v.at[i]],
                                  sem_s[i % NBUF]).wait()

        plsc.subcore_barrier()

        @pl.loop(0, ROWS_PER_TILE // CHUNK)
        def _(j):
            off = base + j * CHUNK
            pltpu.sync_copy(acc.at[pl.ds(off, CHUNK)], rows[0])
            pltpu.sync_copy(rows[0], out_hbm.at[c, pl.ds(off, CHUNK)])

    return body(g, src2, dst3, zeroF)


def _dot(a, b):
    return lax.dot_general(a, b, (((1,), (0,)), ((), ())),
                           precision=_HIGH, preferred_element_type=jnp.float32)


def _acc_spec():
    return pl.BlockSpec((NC, ROW_TILE, FH), lambda i: (0, i, 0))


def _half_spec():
    return pl.BlockSpec((ROW_TILE, FH), lambda i: (i, 0))


def _comb(p_ref, g_ref):
    return p_ref[0] + p_ref[1] + g_ref[...]


def _k_prep(degp, x):
    def body(degp_ref, x_ref, dinv_ref, g0a_ref, g0b_ref):
        deg = degp_ref[0, :, 0:1] + degp_ref[1, :, 0:1] + 1.0
        dinv = lax.rsqrt(deg)
        dinv_ref[...] = dinv
        g0 = dinv * x_ref[...]
        g0a_ref[...] = g0[:, :FH]
        g0b_ref[...] = g0[:, FH:]

    return pl.pallas_call(
        body,
        grid=(GRID,),
        in_specs=[
            pl.BlockSpec((NC, ROW_TILE, 16), lambda i: (0, i, 0)),
            pl.BlockSpec((ROW_TILE, IN_DIM), lambda i: (i, 0)),
        ],
        out_specs=[
            pl.BlockSpec((ROW_TILE, 1), lambda i: (i, 0)),
            _half_spec(),
            _half_spec(),
        ],
        out_shape=[
            jax.ShapeDtypeStruct((N_NODES, 1), jnp.float32),
            jax.ShapeDtypeStruct((N_NODES, FH), jnp.float32),
            jax.ShapeDtypeStruct((N_NODES, FH), jnp.float32),
        ],
    )(degp, x)


def _k_layer1(p0, p1, g0a, g0b, dinv, W1, b1):
    def body(p0_ref, p1_ref, g0a_ref, g0b_ref, dinv_ref, W1_ref, b1_ref,
             ga_ref, gb_ref, gc_ref, gd_ref):
        dinv = dinv_ref[...]
        z = dinv * jnp.concatenate(
            [_comb(p0_ref, g0a_ref), _comb(p1_ref, g0b_ref)], axis=1)
        h = jax.nn.relu(_dot(z, W1_ref[...]) + b1_ref[...])
        ga_ref[...] = dinv * h[:, :FH]
        gb_ref[...] = dinv * h[:, FH:2 * FH]
        gc_ref[...] = dinv * h[:, 2 * FH:3 * FH]
        gd_ref[...] = dinv * h[:, 3 * FH:]

    return pl.pallas_call(
        body,
        grid=(GRID,),
        in_specs=[
            _acc_spec(),
            _acc_spec(),
            _half_spec(),
            _half_spec(),
            pl.BlockSpec((ROW_TILE, 1), lambda i: (i, 0)),
            pl.BlockSpec((IN_DIM, HID_DIM), lambda i: (0, 0)),
            pl.BlockSpec((1, HID_DIM), lambda i: (0, 0)),
        ],
        out_specs=[_half_spec() for _ in range(4)],
        out_shape=[jax.ShapeDtypeStruct((N_NODES, FH), jnp.float32)
                   for _ in range(4)],
    )(p0, p1, g0a, g0b, dinv, W1, b1)


def _k_layer23(q0, q1, q2, q3, g1a, g1b, g1c, g1d, dinv, W2, b2, W3):
    def body(q0_ref, q1_ref, q2_ref, q3_ref, ga_ref, gb_ref, gc_ref, gd_ref,
             dinv_ref, W2_ref, b2_ref, W3_ref, g2a_ref, g2b_ref):
        dinv = dinv_ref[...]
        z = dinv * jnp.concatenate(
            [_comb(q0_ref, ga_ref), _comb(q1_ref, gb_ref),
             _comb(q2_ref, gc_ref), _comb(q3_ref, gd_ref)], axis=1)
        h2 = jax.nn.relu(_dot(z, W2_ref[...]) + b2_ref[...])
        m = _dot(h2, W3_ref[...])
        g2 = dinv * m
        g2a_ref[...] = g2[:, :FH]
        g2b_ref[...] = g2[:, FH:]

    return pl.pallas_call(
        body,
        grid=(GRID,),
        in_specs=[
            _acc_spec(), _acc_spec(), _acc_spec(), _acc_spec(),
            _half_spec(), _half_spec(), _half_spec(), _half_spec(),
            pl.BlockSpec((ROW_TILE, 1), lambda i: (i, 0)),
            pl.BlockSpec((HID_DIM, HID_DIM), lambda i: (0, 0)),
            pl.BlockSpec((1, HID_DIM), lambda i: (0, 0)),
            pl.BlockSpec((HID_DIM, OUT_DIM), lambda i: (0, 0)),
        ],
        out_specs=[_half_spec(), _half_spec()],
        out_shape=[jax.ShapeDtypeStruct((N_NODES, FH), jnp.float32)
                   for _ in range(2)],
    )(q0, q1, q2, q3, g1a, g1b, g1c, g1d, dinv, W2, b2, W3)


def _k_final(r0, r1, g2a, g2b, dinv, b3, batch2, We, be):
    def body(r0_ref, r1_ref, g2a_ref, g2b_ref, dinv_ref, b3_ref, batch_ref,
             We_ref, be_ref, h_ref, emb_ref, pool_acc, cnt_acc):
        i = pl.program_id(0)
        h3 = dinv_ref[...] * jnp.concatenate(
            [_comb(r0_ref, g2a_ref), _comb(r1_ref, g2b_ref)], axis=1)
        h3 = h3 + b3_ref[...]
        h_ref[...] = h3

        seg = lax.broadcasted_iota(jnp.int32, (ROW_TILE, NUM_GRAPHS), 1)
        oh = (batch_ref[...] == seg).astype(jnp.float32)

        @pl.when(i == 0)
        def _():
            pool_acc[...] = jnp.zeros_like(pool_acc)
            cnt_acc[...] = jnp.zeros_like(cnt_acc)

        contract = (((0,), (0,)), ((), ()))
        pool_acc[...] += lax.dot_general(
            oh, h3, contract, precision=_HIGH,
            preferred_element_type=jnp.float32)
        cnt_acc[...] += lax.dot_general(
            oh, jnp.ones((ROW_TILE, OUT_DIM), jnp.float32), contract,
            precision=_HIGH, preferred_element_type=jnp.float32)

        @pl.when(i == GRID - 1)
        def _():
            mean = pool_acc[...] / jnp.maximum(cnt_acc[...], 1.0)
            emb_ref[...] = _dot(mean, We_ref[...]) + be_ref[...]

    return pl.pallas_call(
        body,
        grid=(GRID,),
        in_specs=[
            _acc_spec(), _acc_spec(),
            _half_spec(), _half_spec(),
            pl.BlockSpec((ROW_TILE, 1), lambda i: (i, 0)),
            pl.BlockSpec((1, OUT_DIM), lambda i: (0, 0)),
            pl.BlockSpec((ROW_TILE, 1), lambda i: (i, 0)),
            pl.BlockSpec((OUT_DIM, OUT_DIM), lambda i: (0, 0)),
            pl.BlockSpec((1, OUT_DIM), lambda i: (0, 0)),
        ],
        out_specs=[
            pl.BlockSpec((ROW_TILE, OUT_DIM), lambda i: (i, 0)),
            pl.BlockSpec((NUM_GRAPHS, OUT_DIM), lambda i: (0, 0)),
        ],
        out_shape=[
            jax.ShapeDtypeStruct((N_NODES, OUT_DIM), jnp.float32),
            jax.ShapeDtypeStruct((NUM_GRAPHS, OUT_DIM), jnp.float32),
        ],
        scratch_shapes=[
            pltpu.VMEM((NUM_GRAPHS, OUT_DIM), jnp.float32),
            pltpu.VMEM((NUM_GRAPHS, OUT_DIM), jnp.float32),
        ],
    )(r0, r1, g2a, g2b, dinv, b3, batch2, We, be)


def kernel(x, edge_index, batch, W1, b1, W2, b2, W3, b3, We, be):
    src = edge_index[0].astype(jnp.int32)
    dst = edge_index[1].astype(jnp.int32)
    npad = E_PAD - N_EDGES
    src2 = jnp.concatenate([src, jnp.zeros((npad,), jnp.int32)])
    src2 = src2.reshape(E_PAD // CHUNK, CHUNK)
    dst3 = jnp.concatenate(
        [dst, jnp.full((npad,), N_NODES, jnp.int32)])
    dst3 = dst3.reshape(E_PAD // CHUNK, CHUNK)
    batch2 = batch.astype(jnp.int32).reshape(N_NODES, 1)
    b1r = b1.reshape(1, HID_DIM)
    b2r = b2.reshape(1, HID_DIM)
    b3r = b3.reshape(1, OUT_DIM)
    ber = be.reshape(1, OUT_DIM)
    e016 = jnp.tile((jnp.arange(16) == 0).astype(jnp.float32), (CHUNK, 1))
    zero16 = jnp.zeros((CHUNK, 16), jnp.float32)
    zeroF = jnp.zeros((CHUNK, FH), jnp.float32)

    degp = _sc_degree(dst3, e016, zero16)
    dinv, g0a, g0b = _k_prep(degp, x)
    p0 = _sc_edge_pass(g0a, src2, dst3, zeroF)
    p1 = _sc_edge_pass(g0b, src2, dst3, zeroF)
    g1a, g1b, g1c, g1d = _k_layer1(p0, p1, g0a, g0b, dinv, W1, b1r)
    q0 = _sc_edge_pass(g1a, src2, dst3, zeroF)
    q1 = _sc_edge_pass(g1b, src2, dst3, zeroF)
    q2 = _sc_edge_pass(g1c, src2, dst3, zeroF)
    q3 = _sc_edge_pass(g1d, src2, dst3, zeroF)
    g2a, g2b = _k_layer23(q0, q1, q2, q3, g1a, g1b, g1c, g1d,
                          dinv, W2, b2r, W3)
    r0 = _sc_edge_pass(g2a, src2, dst3, zeroF)
    r1 = _sc_edge_pass(g2b, src2, dst3, zeroF)
    h, emb = _k_final(r0, r1, g2a, g2b, dinv, b3r, batch2, We, ber)
    return (emb, h)


# speedup vs baseline: 6.7287x; 1.0227x over previous
import functools

import jax
import jax.numpy as jnp
from jax import lax
from jax.experimental import pallas as pl
from jax.experimental.pallas import tpu as pltpu
from jax.experimental.pallas import tpu_sc as plsc

N_NODES = 10000
N_EDGES = 320000
IN_DIM = 128
HID_DIM = 256
OUT_DIM = 128
NUM_GRAPHS = 64

NC = 2
NS = 16

E_PER_TILE = 10240
E_PAD = E_PER_TILE * NC * NS
CHUNK = 256
IR = CHUNK // 128
N_CHUNKS = E_PER_TILE // CHUNK
N_ACC = 10240
ROWS_PER_TILE = N_ACC // NS
FH = 64
NBUF = 4
AHEAD = 2

ROW_TILE = 1000
GRID = N_NODES // ROW_TILE

_HIGH = lax.Precision.HIGHEST


def _sc_degree(dst3, e016, zero16):
    mesh = plsc.VectorSubcoreMesh(core_axis_name="c", subcore_axis_name="s")

    @functools.partial(
        pl.kernel,
        mesh=mesh,
        compiler_params=pltpu.CompilerParams(use_tc_tiling_on_sc=False),
        out_type=jax.ShapeDtypeStruct((NC, N_ACC, 16), jnp.float32),
        scratch_types=[
            pltpu.VMEM((E_PER_TILE,), jnp.int32),
            pltpu.VMEM((CHUNK, 16), jnp.float32),
            pltpu.VMEM_SHARED((N_ACC, 16), jnp.float32),
            pltpu.SemaphoreType.DMA,
        ],
    )
    def body(dst_hbm, e0_hbm, z_hbm, out_hbm, dst_v, val, acc, sem):
        c = lax.axis_index("c")
        s = lax.axis_index("s")
        t = c * NS + s

        base = s * ROWS_PER_TILE
        pltpu.sync_copy(z_hbm, val)

        @pl.loop(0, ROWS_PER_TILE // 128)
        def _(j):
            pltpu.sync_copy(val.at[pl.ds(0, 128)],
                            acc.at[pl.ds(base + j * 128, 128)])

        pltpu.sync_copy(e0_hbm, val)
        pltpu.sync_copy(dst_hbm.at[pl.ds(t * E_PER_TILE, E_PER_TILE)], dst_v)
        plsc.subcore_barrier()

        @pl.loop(0, N_CHUNKS)
        def _(i):
            pltpu.async_copy(val, acc.at[dst_v.at[pl.ds(i * CHUNK, CHUNK)]],
                             sem, add=True).wait()

        plsc.subcore_barrier()

        @pl.loop(0, ROWS_PER_TILE // 128)
        def _(j):
            off = base + j * 128
            pltpu.sync_copy(acc.at[pl.ds(off, 128)], val.at[pl.ds(0, 128)])
            pltpu.sync_copy(val.at[pl.ds(0, 128)],
                            out_hbm.at[c, pl.ds(off, 128)])

    return body(dst3, e016, zero16)


def _sc_edge_pass(g, src2, dst3, zeroF):
    mesh = plsc.VectorSubcoreMesh(core_axis_name="c", subcore_axis_name="s")

    @functools.partial(
        pl.kernel,
        mesh=mesh,
        compiler_params=pltpu.CompilerParams(use_tc_tiling_on_sc=False),
        out_type=jax.ShapeDtypeStruct((NC, N_ACC, FH), jnp.float32),
        scratch_types=[
            pltpu.VMEM((E_PER_TILE,), jnp.int32),
            pltpu.VMEM((E_PER_TILE,), jnp.int32),
            [pltpu.VMEM((CHUNK, FH), jnp.float32) for _ in range(NBUF)],
            [pltpu.SemaphoreType.DMA for _ in range(NBUF)],
            [pltpu.SemaphoreType.DMA for _ in range(NBUF)],
            pltpu.VMEM_SHARED((N_ACC, FH), jnp.float32),
        ],
    )
    def body(g_hbm, src_hbm, dst_hbm, z_hbm, out_hbm,
             src_v, dst_v, rows, sem_g, sem_s, acc):
        c = lax.axis_index("c")
        s = lax.axis_index("s")
        t = c * NS + s

        pltpu.sync_copy(z_hbm, rows[0])
        base = s * ROWS_PER_TILE

        @pl.loop(0, ROWS_PER_TILE // 128)
        def _(j):
            pltpu.sync_copy(rows[0].at[pl.ds(0, 128)],
                            acc.at[pl.ds(base + j * 128, 128)])

        pltpu.sync_copy(src_hbm.at[pl.ds(t * E_PER_TILE, E_PER_TILE)], src_v)
        pltpu.sync_copy(dst_hbm.at[pl.ds(t * E_PER_TILE, E_PER_TILE)], dst_v)
        plsc.subcore_barrier()

        for b in range(AHEAD):
            pltpu.async_copy(g_hbm.at[src_v.at[pl.ds(b * CHUNK, CHUNK)]], rows[b], sem_g[b])

        @pl.loop(0, N_CHUNKS // NBUF)
        def _(o):
            for b in range(NBUF):
                i = o * NBUF + b
                j_b = (b + AHEAD) % NBUF

                def prefetch(i=i, b=b, j_b=j_b):
                    j = i + AHEAD

                    def drain():
                        pltpu.make_async_copy(
                            rows[j_b], acc.at[dst_v.at[pl.ds((j - NBUF) * CHUNK, CHUNK)]],
                            sem_s[j_b]).wait()

                    if b + AHEAD >= NBUF:
                        drain()
                    else:
                        @pl.when(o >= 1)
                        def _():
                            drain()
                    pltpu.async_copy(
                        g_hbm.at[src_v.at[pl.ds(j * CHUNK, CHUNK)]],
                        rows[j_b], sem_g[j_b])

                if b + AHEAD < NBUF:
                    prefetch()
                else:
                    @pl.when(o < N_CHUNKS // NBUF - 1)
                    def _():
                        prefetch()

                pltpu.make_async_copy(
                    g_hbm.at[src_v.at[pl.ds(i * CHUNK, CHUNK)]],
                    rows[b], sem_g[b]).wait()
                pltpu.async_copy(rows[b],
                                 acc.at[dst_v.at[pl.ds(i * CHUNK, CHUNK)]],
                                 sem_s[b], add=True)

        for k in range(NBUF):
            i = N_CHUNKS - NBUF + k
            pltpu.make_async_copy(
                rows[i % NBUF], acc.at[dst_v.at[pl.ds(i * CHUNK, CHUNK)]],
                sem_s[i % NBUF]).wait()

        plsc.subcore_barrier()

        @pl.loop(0, ROWS_PER_TILE // 128)
        def _(j):
            off = base + j * 128
            pltpu.sync_copy(acc.at[pl.ds(off, 128)], rows[0].at[pl.ds(0, 128)])
            pltpu.sync_copy(rows[0].at[pl.ds(0, 128)],
                            out_hbm.at[c, pl.ds(off, 128)])

    return body(g, src2, dst3, zeroF)


def _dot(a, b):
    return lax.dot_general(a, b, (((1,), (0,)), ((), ())),
                           precision=_HIGH, preferred_element_type=jnp.float32)


def _acc_spec():
    return pl.BlockSpec((NC, ROW_TILE, FH), lambda i: (0, i, 0))


def _half_spec():
    return pl.BlockSpec((ROW_TILE, FH), lambda i: (i, 0))


def _comb(p_ref, g_ref):
    return p_ref[0] + p_ref[1] + g_ref[...]


def _k_prep(degp, x):
    def body(degp_ref, x_ref, dinv_ref, g0a_ref, g0b_ref):
        deg = degp_ref[0, :, 0:1] + degp_ref[1, :, 0:1] + 1.0
        dinv = lax.rsqrt(deg)
        dinv_ref[...] = dinv
        g0 = dinv * x_ref[...]
        g0a_ref[...] = g0[:, :FH]
        g0b_ref[...] = g0[:, FH:]

    return pl.pallas_call(
        body,
        grid=(GRID,),
        in_specs=[
            pl.BlockSpec((NC, ROW_TILE, 16), lambda i: (0, i, 0)),
            pl.BlockSpec((ROW_TILE, IN_DIM), lambda i: (i, 0)),
        ],
        out_specs=[
            pl.BlockSpec((ROW_TILE, 1), lambda i: (i, 0)),
            _half_spec(),
            _half_spec(),
        ],
        out_shape=[
            jax.ShapeDtypeStruct((N_NODES, 1), jnp.float32),
            jax.ShapeDtypeStruct((N_NODES, FH), jnp.float32),
            jax.ShapeDtypeStruct((N_NODES, FH), jnp.float32),
        ],
    )(degp, x)


def _k_layer1(p0, p1, g0a, g0b, dinv, W1, b1):
    def body(p0_ref, p1_ref, g0a_ref, g0b_ref, dinv_ref, W1_ref, b1_ref,
             ga_ref, gb_ref, gc_ref, gd_ref):
        dinv = dinv_ref[...]
        z = dinv * jnp.concatenate(
            [_comb(p0_ref, g0a_ref), _comb(p1_ref, g0b_ref)], axis=1)
        h = jax.nn.relu(_dot(z, W1_ref[...]) + b1_ref[...])
        ga_ref[...] = dinv * h[:, :FH]
        gb_ref[...] = dinv * h[:, FH:2 * FH]
        gc_ref[...] = dinv * h[:, 2 * FH:3 * FH]
        gd_ref[...] = dinv * h[:, 3 * FH:]

    return pl.pallas_call(
        body,
        grid=(GRID,),
        in_specs=[
            _acc_spec(),
            _acc_spec(),
            _half_spec(),
            _half_spec(),
            pl.BlockSpec((ROW_TILE, 1), lambda i: (i, 0)),
            pl.BlockSpec((IN_DIM, HID_DIM), lambda i: (0, 0)),
            pl.BlockSpec((1, HID_DIM), lambda i: (0, 0)),
        ],
        out_specs=[_half_spec() for _ in range(4)],
        out_shape=[jax.ShapeDtypeStruct((N_NODES, FH), jnp.float32)
                   for _ in range(4)],
    )(p0, p1, g0a, g0b, dinv, W1, b1)


def _k_layer23(q0, q1, q2, q3, g1a, g1b, g1c, g1d, dinv, W2, b2, W3):
    def body(q0_ref, q1_ref, q2_ref, q3_ref, ga_ref, gb_ref, gc_ref, gd_ref,
             dinv_ref, W2_ref, b2_ref, W3_ref, g2a_ref, g2b_ref):
        dinv = dinv_ref[...]
        z = dinv * jnp.concatenate(
            [_comb(q0_ref, ga_ref), _comb(q1_ref, gb_ref),
             _comb(q2_ref, gc_ref), _comb(q3_ref, gd_ref)], axis=1)
        h2 = jax.nn.relu(_dot(z, W2_ref[...]) + b2_ref[...])
        m = _dot(h2, W3_ref[...])
        g2 = dinv * m
        g2a_ref[...] = g2[:, :FH]
        g2b_ref[...] = g2[:, FH:]

    return pl.pallas_call(
        body,
        grid=(GRID,),
        in_specs=[
            _acc_spec(), _acc_spec(), _acc_spec(), _acc_spec(),
            _half_spec(), _half_spec(), _half_spec(), _half_spec(),
            pl.BlockSpec((ROW_TILE, 1), lambda i: (i, 0)),
            pl.BlockSpec((HID_DIM, HID_DIM), lambda i: (0, 0)),
            pl.BlockSpec((1, HID_DIM), lambda i: (0, 0)),
            pl.BlockSpec((HID_DIM, OUT_DIM), lambda i: (0, 0)),
        ],
        out_specs=[_half_spec(), _half_spec()],
        out_shape=[jax.ShapeDtypeStruct((N_NODES, FH), jnp.float32)
                   for _ in range(2)],
    )(q0, q1, q2, q3, g1a, g1b, g1c, g1d, dinv, W2, b2, W3)


def _k_final(r0, r1, g2a, g2b, dinv, b3, batch2, We, be):
    def body(r0_ref, r1_ref, g2a_ref, g2b_ref, dinv_ref, b3_ref, batch_ref,
             We_ref, be_ref, h_ref, emb_ref, pool_acc, cnt_acc):
        i = pl.program_id(0)
        h3 = dinv_ref[...] * jnp.concatenate(
            [_comb(r0_ref, g2a_ref), _comb(r1_ref, g2b_ref)], axis=1)
        h3 = h3 + b3_ref[...]
        h_ref[...] = h3

        seg = lax.broadcasted_iota(jnp.int32, (ROW_TILE, NUM_GRAPHS), 1)
        oh = (batch_ref[...] == seg).astype(jnp.float32)

        @pl.when(i == 0)
        def _():
            pool_acc[...] = jnp.zeros_like(pool_acc)
            cnt_acc[...] = jnp.zeros_like(cnt_acc)

        contract = (((0,), (0,)), ((), ()))
        pool_acc[...] += lax.dot_general(
            oh, h3, contract, precision=_HIGH,
            preferred_element_type=jnp.float32)
        cnt_acc[...] += lax.dot_general(
            oh, jnp.ones((ROW_TILE, OUT_DIM), jnp.float32), contract,
            precision=_HIGH, preferred_element_type=jnp.float32)

        @pl.when(i == GRID - 1)
        def _():
            mean = pool_acc[...] / jnp.maximum(cnt_acc[...], 1.0)
            emb_ref[...] = _dot(mean, We_ref[...]) + be_ref[...]

    return pl.pallas_call(
        body,
        grid=(GRID,),
        in_specs=[
            _acc_spec(), _acc_spec(),
            _half_spec(), _half_spec(),
            pl.BlockSpec((ROW_TILE, 1), lambda i: (i, 0)),
            pl.BlockSpec((1, OUT_DIM), lambda i: (0, 0)),
            pl.BlockSpec((ROW_TILE, 1), lambda i: (i, 0)),
            pl.BlockSpec((OUT_DIM, OUT_DIM), lambda i: (0, 0)),
            pl.BlockSpec((1, OUT_DIM), lambda i: (0, 0)),
        ],
        out_specs=[
            pl.BlockSpec((ROW_TILE, OUT_DIM), lambda i: (i, 0)),
            pl.BlockSpec((NUM_GRAPHS, OUT_DIM), lambda i: (0, 0)),
        ],
        out_shape=[
            jax.ShapeDtypeStruct((N_NODES, OUT_DIM), jnp.float32),
            jax.ShapeDtypeStruct((NUM_GRAPHS, OUT_DIM), jnp.float32),
        ],
        scratch_shapes=[
            pltpu.VMEM((NUM_GRAPHS, OUT_DIM), jnp.float32),
            pltpu.VMEM((NUM_GRAPHS, OUT_DIM), jnp.float32),
        ],
    )(r0, r1, g2a, g2b, dinv, b3, batch2, We, be)


def kernel(x, edge_index, batch, W1, b1, W2, b2, W3, b3, We, be):
    src = edge_index[0].astype(jnp.int32)
    dst = edge_index[1].astype(jnp.int32)
    npad = E_PAD - N_EDGES
    src2 = jnp.concatenate([src, jnp.zeros((npad,), jnp.int32)])
    dst3 = jnp.concatenate(
        [dst, jnp.full((npad,), N_NODES, jnp.int32)])
    batch2 = batch.astype(jnp.int32).reshape(N_NODES, 1)
    b1r = b1.reshape(1, HID_DIM)
    b2r = b2.reshape(1, HID_DIM)
    b3r = b3.reshape(1, OUT_DIM)
    ber = be.reshape(1, OUT_DIM)
    e016 = jnp.tile((jnp.arange(16) == 0).astype(jnp.float32), (CHUNK, 1))
    zero16 = jnp.zeros((CHUNK, 16), jnp.float32)
    zeroF = jnp.zeros((CHUNK, FH), jnp.float32)

    degp = _sc_degree(dst3, e016, zero16)
    dinv, g0a, g0b = _k_prep(degp, x)
    p0 = _sc_edge_pass(g0a, src2, dst3, zeroF)
    p1 = _sc_edge_pass(g0b, src2, dst3, zeroF)
    g1a, g1b, g1c, g1d = _k_layer1(p0, p1, g0a, g0b, dinv, W1, b1r)
    q0 = _sc_edge_pass(g1a, src2, dst3, zeroF)
    q1 = _sc_edge_pass(g1b, src2, dst3, zeroF)
    q2 = _sc_edge_pass(g1c, src2, dst3, zeroF)
    q3 = _sc_edge_pass(g1d, src2, dst3, zeroF)
    g2a, g2b = _k_layer23(q0, q1, q2, q3, g1a, g1b, g1c, g1d,
                          dinv, W2, b2r, W3)
    r0 = _sc_edge_pass(g2a, src2, dst3, zeroF)
    r1 = _sc_edge_pass(g2b, src2, dst3, zeroF)
    h, emb = _k_final(r0, r1, g2a, g2b, dinv, b3r, batch2, We, ber)
    return (emb, h)


# speedup vs baseline: 11.8469x; 1.7607x over previous
import functools

import jax
import jax.numpy as jnp
from jax import lax
from jax.experimental import pallas as pl
from jax.experimental.pallas import tpu as pltpu
from jax.experimental.pallas import tpu_sc as plsc

N_NODES = 10000
N_EDGES = 320000
IN_DIM = 128
HID_DIM = 256
OUT_DIM = 128
NUM_GRAPHS = 64

NC = 2
NS = 16

E_PER_TILE = 10240
E_PAD = E_PER_TILE * NC * NS
CHUNK = 256
IR = CHUNK // 128
N_CHUNKS = E_PER_TILE // CHUNK
N_ACC = 10240
ROWS_PER_TILE = N_ACC // NS
FW = 128
NBUF = 4
AHEAD = 2

ROW_TILE = 1000
GRID = N_NODES // ROW_TILE

_HIGH = lax.Precision.HIGHEST


def _sc_degree(dst3, e016, zero16):
    mesh = plsc.VectorSubcoreMesh(core_axis_name="c", subcore_axis_name="s")

    @functools.partial(
        pl.kernel,
        mesh=mesh,
        compiler_params=pltpu.CompilerParams(use_tc_tiling_on_sc=False),
        out_type=jax.ShapeDtypeStruct((NC, N_ACC, 16), jnp.float32),
        scratch_types=[
            pltpu.VMEM((E_PER_TILE,), jnp.int32),
            pltpu.VMEM((CHUNK, 16), jnp.float32),
            pltpu.VMEM_SHARED((N_ACC, 16), jnp.float32),
            pltpu.SemaphoreType.DMA,
        ],
    )
    def body(dst_hbm, e0_hbm, z_hbm, out_hbm, dst_v, val, acc, sem):
        c = lax.axis_index("c")
        s = lax.axis_index("s")
        t = c * NS + s

        base = s * ROWS_PER_TILE
        pltpu.sync_copy(z_hbm, val)

        @pl.loop(0, ROWS_PER_TILE // 128)
        def _(j):
            pltpu.sync_copy(val.at[pl.ds(0, 128)],
                            acc.at[pl.ds(base + j * 128, 128)])

        pltpu.sync_copy(e0_hbm, val)
        pltpu.sync_copy(dst_hbm.at[pl.ds(t * E_PER_TILE, E_PER_TILE)], dst_v)
        plsc.subcore_barrier()

        @pl.loop(0, N_CHUNKS)
        def _(i):
            pltpu.async_copy(val, acc.at[dst_v.at[pl.ds(i * CHUNK, CHUNK)]],
                             sem, add=True).wait()

        plsc.subcore_barrier()

        @pl.loop(0, ROWS_PER_TILE // 128)
        def _(j):
            off = base + j * 128
            pltpu.sync_copy(acc.at[pl.ds(off, 128)], val.at[pl.ds(0, 128)])
            pltpu.sync_copy(val.at[pl.ds(0, 128)],
                            out_hbm.at[c, pl.ds(off, 128)])

    return body(dst3, e016, zero16)


def _sc_edge_pass(g, src2, dst3, zeroF):
    mesh = plsc.VectorSubcoreMesh(core_axis_name="c", subcore_axis_name="s")

    @functools.partial(
        pl.kernel,
        mesh=mesh,
        compiler_params=pltpu.CompilerParams(use_tc_tiling_on_sc=False),
        out_type=jax.ShapeDtypeStruct((NC, N_ACC, FW), jnp.bfloat16),
        scratch_types=[
            pltpu.VMEM((E_PER_TILE,), jnp.int32),
            pltpu.VMEM((E_PER_TILE,), jnp.int32),
            [pltpu.VMEM((CHUNK, FW), jnp.bfloat16) for _ in range(NBUF)],
            [pltpu.SemaphoreType.DMA for _ in range(NBUF)],
            [pltpu.SemaphoreType.DMA for _ in range(NBUF)],
            pltpu.VMEM_SHARED((N_ACC, FW), jnp.bfloat16),
        ],
    )
    def body(g_hbm, src_hbm, dst_hbm, z_hbm, out_hbm,
             src_v, dst_v, rows, sem_g, sem_s, acc):
        c = lax.axis_index("c")
        s = lax.axis_index("s")
        t = c * NS + s

        pltpu.sync_copy(z_hbm, rows[0])
        base = s * ROWS_PER_TILE

        @pl.loop(0, ROWS_PER_TILE // 128)
        def _(j):
            pltpu.sync_copy(rows[0].at[pl.ds(0, 128)],
                            acc.at[pl.ds(base + j * 128, 128)])

        pltpu.sync_copy(src_hbm.at[pl.ds(t * E_PER_TILE, E_PER_TILE)], src_v)
        pltpu.sync_copy(dst_hbm.at[pl.ds(t * E_PER_TILE, E_PER_TILE)], dst_v)
        plsc.subcore_barrier()

        for b in range(AHEAD):
            pltpu.async_copy(g_hbm.at[src_v.at[pl.ds(b * CHUNK, CHUNK)]], rows[b], sem_g[b])

        @pl.loop(0, N_CHUNKS // NBUF)
        def _(o):
            for b in range(NBUF):
                i = o * NBUF + b
                j_b = (b + AHEAD) % NBUF

                def prefetch(i=i, b=b, j_b=j_b):
                    j = i + AHEAD

                    def drain():
                        pltpu.make_async_copy(
                            rows[j_b], acc.at[dst_v.at[pl.ds((j - NBUF) * CHUNK, CHUNK)]],
                            sem_s[j_b]).wait()

                    if b + AHEAD >= NBUF:
                        drain()
                    else:
                        @pl.when(o >= 1)
                        def _():
                            drain()
                    pltpu.async_copy(
                        g_hbm.at[src_v.at[pl.ds(j * CHUNK, CHUNK)]],
                        rows[j_b], sem_g[j_b])

                if b + AHEAD < NBUF:
                    prefetch()
                else:
                    @pl.when(o < N_CHUNKS // NBUF - 1)
                    def _():
                        prefetch()

                pltpu.make_async_copy(
                    g_hbm.at[src_v.at[pl.ds(i * CHUNK, CHUNK)]],
                    rows[b], sem_g[b]).wait()
                pltpu.async_copy(rows[b],
                                 acc.at[dst_v.at[pl.ds(i * CHUNK, CHUNK)]],
                                 sem_s[b], add=True)

        for k in range(NBUF):
            i = N_CHUNKS - NBUF + k
            pltpu.make_async_copy(
                rows[i % NBUF], acc.at[dst_v.at[pl.ds(i * CHUNK, CHUNK)]],
                sem_s[i % NBUF]).wait()

        plsc.subcore_barrier()

        @pl.loop(0, ROWS_PER_TILE // 128)
        def _(j):
            off = base + j * 128
            pltpu.sync_copy(acc.at[pl.ds(off, 128)], rows[0].at[pl.ds(0, 128)])
            pltpu.sync_copy(rows[0].at[pl.ds(0, 128)],
                            out_hbm.at[c, pl.ds(off, 128)])

    return body(g, src2, dst3, zeroF)


def _dot(a, b):
    return lax.dot_general(a, b, (((1,), (0,)), ((), ())),
                           precision=_HIGH, preferred_element_type=jnp.float32)


def _acc_spec():
    return pl.BlockSpec((NC, ROW_TILE, FW), lambda i: (0, i, 0))


def _full_spec():
    return pl.BlockSpec((ROW_TILE, FW), lambda i: (i, 0))


def _psum(p_ref):
    return (p_ref[0].astype(jnp.float32) + p_ref[1].astype(jnp.float32))


def _k_prep(degp, x):
    def body(degp_ref, x_ref, dinv_ref, g0_ref):
        deg = degp_ref[0, :, 0:1] + degp_ref[1, :, 0:1] + 1.0
        dinv = lax.rsqrt(deg)
        dinv_ref[...] = dinv
        g0_ref[...] = (dinv * x_ref[...]).astype(jnp.bfloat16)

    return pl.pallas_call(
        body,
        grid=(GRID,),
        in_specs=[
            pl.BlockSpec((NC, ROW_TILE, 16), lambda i: (0, i, 0)),
            pl.BlockSpec((ROW_TILE, IN_DIM), lambda i: (i, 0)),
        ],
        out_specs=[
            pl.BlockSpec((ROW_TILE, 1), lambda i: (i, 0)),
            _full_spec(),
        ],
        out_shape=[
            jax.ShapeDtypeStruct((N_NODES, 1), jnp.float32),
            jax.ShapeDtypeStruct((N_NODES, FW), jnp.bfloat16),
        ],
    )(degp, x)


def _k_layer1(p, x, dinv, W1, b1):
    def body(p_ref, x_ref, dinv_ref, W1_ref, b1_ref,
             h1_ref, g1a_ref, g1b_ref):
        dinv = dinv_ref[...]
        z = dinv * _psum(p_ref) + (dinv * dinv) * x_ref[...]
        h = jax.nn.relu(_dot(z, W1_ref[...]) + b1_ref[...])
        h1_ref[...] = h
        g1a_ref[...] = (dinv * h[:, :FW]).astype(jnp.bfloat16)
        g1b_ref[...] = (dinv * h[:, FW:]).astype(jnp.bfloat16)

    return pl.pallas_call(
        body,
        grid=(GRID,),
        in_specs=[
            _acc_spec(),
            pl.BlockSpec((ROW_TILE, IN_DIM), lambda i: (i, 0)),
            pl.BlockSpec((ROW_TILE, 1), lambda i: (i, 0)),
            pl.BlockSpec((IN_DIM, HID_DIM), lambda i: (0, 0)),
            pl.BlockSpec((1, HID_DIM), lambda i: (0, 0)),
        ],
        out_specs=[
            pl.BlockSpec((ROW_TILE, HID_DIM), lambda i: (i, 0)),
            _full_spec(),
            _full_spec(),
        ],
        out_shape=[
            jax.ShapeDtypeStruct((N_NODES, HID_DIM), jnp.float32),
            jax.ShapeDtypeStruct((N_NODES, FW), jnp.bfloat16),
            jax.ShapeDtypeStruct((N_NODES, FW), jnp.bfloat16),
        ],
    )(p, x, dinv, W1, b1)


def _k_layer23(qa, qb, h1, dinv, W2, b2, W3):
    def body(qa_ref, qb_ref, h1_ref, dinv_ref, W2_ref, b2_ref, W3_ref,
             m_ref, g2_ref):
        dinv = dinv_ref[...]
        d2 = dinv * dinv
        h1 = h1_ref[...]
        z0 = dinv * _psum(qa_ref) + d2 * h1[:, :FW]
        z1 = dinv * _psum(qb_ref) + d2 * h1[:, FW:]
        h2 = jax.nn.relu(_dot(z0, W2_ref[:FW, :]) + _dot(z1, W2_ref[FW:, :])
                         + b2_ref[...])
        m = _dot(h2, W3_ref[...])
        m_ref[...] = m
        g2_ref[...] = (dinv * m).astype(jnp.bfloat16)

    return pl.pallas_call(
        body,
        grid=(GRID,),
        in_specs=[
            _acc_spec(), _acc_spec(),
            pl.BlockSpec((ROW_TILE, HID_DIM), lambda i: (i, 0)),
            pl.BlockSpec((ROW_TILE, 1), lambda i: (i, 0)),
            pl.BlockSpec((HID_DIM, HID_DIM), lambda i: (0, 0)),
            pl.BlockSpec((1, HID_DIM), lambda i: (0, 0)),
            pl.BlockSpec((HID_DIM, OUT_DIM), lambda i: (0, 0)),
        ],
        out_specs=[
            pl.BlockSpec((ROW_TILE, OUT_DIM), lambda i: (i, 0)),
            _full_spec(),
        ],
        out_shape=[
            jax.ShapeDtypeStruct((N_NODES, OUT_DIM), jnp.float32),
            jax.ShapeDtypeStruct((N_NODES, FW), jnp.bfloat16),
        ],
    )(qa, qb, h1, dinv, W2, b2, W3)


def _k_final(r, m, dinv, b3, batch2, We, be):
    def body(r_ref, m_ref, dinv_ref, b3_ref, batch_ref,
             We_ref, be_ref, h_ref, emb_ref, pool_acc, cnt_acc):
        i = pl.program_id(0)
        dinv = dinv_ref[...]
        h3 = dinv * _psum(r_ref) + (dinv * dinv) * m_ref[...] + b3_ref[...]
        h_ref[...] = h3

        seg = lax.broadcasted_iota(jnp.int32, (ROW_TILE, NUM_GRAPHS), 1)
        oh = (batch_ref[...] == seg).astype(jnp.float32)

        @pl.when(i == 0)
        def _():
            pool_acc[...] = jnp.zeros_like(pool_acc)
            cnt_acc[...] = jnp.zeros_like(cnt_acc)

        contract = (((0,), (0,)), ((), ()))
        pool_acc[...] += lax.dot_general(
            oh, h3, contract, precision=_HIGH,
            preferred_element_type=jnp.float32)
        cnt_acc[...] += lax.dot_general(
            oh, jnp.ones((ROW_TILE, OUT_DIM), jnp.float32), contract,
            precision=_HIGH, preferred_element_type=jnp.float32)

        @pl.when(i == GRID - 1)
        def _():
            mean = pool_acc[...] / jnp.maximum(cnt_acc[...], 1.0)
            emb_ref[...] = _dot(mean, We_ref[...]) + be_ref[...]

    return pl.pallas_call(
        body,
        grid=(GRID,),
        in_specs=[
            _acc_spec(),
            pl.BlockSpec((ROW_TILE, OUT_DIM), lambda i: (i, 0)),
            pl.BlockSpec((ROW_TILE, 1), lambda i: (i, 0)),
            pl.BlockSpec((1, OUT_DIM), lambda i: (0, 0)),
            pl.BlockSpec((ROW_TILE, 1), lambda i: (i, 0)),
            pl.BlockSpec((OUT_DIM, OUT_DIM), lambda i: (0, 0)),
            pl.BlockSpec((1, OUT_DIM), lambda i: (0, 0)),
        ],
        out_specs=[
            pl.BlockSpec((ROW_TILE, OUT_DIM), lambda i: (i, 0)),
            pl.BlockSpec((NUM_GRAPHS, OUT_DIM), lambda i: (0, 0)),
        ],
        out_shape=[
            jax.ShapeDtypeStruct((N_NODES, OUT_DIM), jnp.float32),
            jax.ShapeDtypeStruct((NUM_GRAPHS, OUT_DIM), jnp.float32),
        ],
        scratch_shapes=[
            pltpu.VMEM((NUM_GRAPHS, OUT_DIM), jnp.float32),
            pltpu.VMEM((NUM_GRAPHS, OUT_DIM), jnp.float32),
        ],
    )(r, m, dinv, b3, batch2, We, be)


def kernel(x, edge_index, batch, W1, b1, W2, b2, W3, b3, We, be):
    src = edge_index[0].astype(jnp.int32)
    dst = edge_index[1].astype(jnp.int32)
    npad = E_PAD - N_EDGES
    src2 = jnp.concatenate([src, jnp.zeros((npad,), jnp.int32)])
    dst3 = jnp.concatenate(
        [dst, jnp.full((npad,), N_NODES, jnp.int32)])
    batch2 = batch.astype(jnp.int32).reshape(N_NODES, 1)
    b1r = b1.reshape(1, HID_DIM)
    b2r = b2.reshape(1, HID_DIM)
    b3r = b3.reshape(1, OUT_DIM)
    ber = be.reshape(1, OUT_DIM)
    e016 = jnp.tile((jnp.arange(16) == 0).astype(jnp.float32), (CHUNK, 1))
    zero16 = jnp.zeros((CHUNK, 16), jnp.float32)
    zeroF = jnp.zeros((CHUNK, FW), jnp.bfloat16)

    degp = _sc_degree(dst3, e016, zero16)
    dinv, g0 = _k_prep(degp, x)
    p = _sc_edge_pass(g0, src2, dst3, zeroF)
    h1, g1a, g1b = _k_layer1(p, x, dinv, W1, b1r)
    qa = _sc_edge_pass(g1a, src2, dst3, zeroF)
    qb = _sc_edge_pass(g1b, src2, dst3, zeroF)
    m, g2 = _k_layer23(qa, qb, h1, dinv, W2, b2r, W3)
    r = _sc_edge_pass(g2, src2, dst3, zeroF)
    h, emb = _k_final(r, m, dinv, b3r, batch2, We, ber)
    return (emb, h)


# speedup vs baseline: 12.0012x; 1.0130x over previous
import functools

import jax
import jax.numpy as jnp
from jax import lax
from jax.experimental import pallas as pl
from jax.experimental.pallas import tpu as pltpu
from jax.experimental.pallas import tpu_sc as plsc

N_NODES = 10000
N_EDGES = 320000
IN_DIM = 128
HID_DIM = 256
OUT_DIM = 128
NUM_GRAPHS = 64

NC = 2
NS = 16

E_PER_TILE = 10240
E_PAD = E_PER_TILE * NC * NS
CHUNK = 256
IR = CHUNK // 128
N_CHUNKS = E_PER_TILE // CHUNK
N_ACC = 10240
ROWS_PER_TILE = N_ACC // NS
FW = 128
NBUF = 4
AHEAD = 2

ROW_TILE = 1000
GRID = N_NODES // ROW_TILE

_HIGH = lax.Precision.HIGHEST


def _sc_degree(dst3, e016, zero16):
    mesh = plsc.VectorSubcoreMesh(core_axis_name="c", subcore_axis_name="s")

    @functools.partial(
        pl.kernel,
        mesh=mesh,
        compiler_params=pltpu.CompilerParams(use_tc_tiling_on_sc=False),
        out_type=jax.ShapeDtypeStruct((NC, N_ACC, 16), jnp.float32),
        scratch_types=[
            pltpu.VMEM((E_PER_TILE,), jnp.int32),
            pltpu.VMEM((CHUNK, 16), jnp.float32),
            pltpu.VMEM_SHARED((N_ACC, 16), jnp.float32),
            pltpu.SemaphoreType.DMA,
        ],
    )
    def body(dst_hbm, e0_hbm, z_hbm, out_hbm, dst_v, val, acc, sem):
        c = lax.axis_index("c")
        s = lax.axis_index("s")
        t = c * NS + s

        base = s * ROWS_PER_TILE
        pltpu.sync_copy(z_hbm, val)

        @pl.loop(0, ROWS_PER_TILE // 128)
        def _(j):
            pltpu.sync_copy(val.at[pl.ds(0, 128)],
                            acc.at[pl.ds(base + j * 128, 128)])

        pltpu.sync_copy(e0_hbm, val)
        pltpu.sync_copy(dst_hbm.at[pl.ds(t * E_PER_TILE, E_PER_TILE)], dst_v)
        plsc.subcore_barrier()

        @pl.loop(0, N_CHUNKS)
        def _(i):
            pltpu.async_copy(val, acc.at[dst_v.at[pl.ds(i * CHUNK, CHUNK)]],
                             sem, add=True)

        @pl.loop(0, N_CHUNKS)
        def _(i):
            pltpu.make_async_copy(
                val, acc.at[dst_v.at[pl.ds(i * CHUNK, CHUNK)]], sem).wait()

        plsc.subcore_barrier()

        @pl.loop(0, ROWS_PER_TILE // 128)
        def _(j):
            off = base + j * 128
            pltpu.sync_copy(acc.at[pl.ds(off, 128)], val.at[pl.ds(0, 128)])
            pltpu.sync_copy(val.at[pl.ds(0, 128)],
                            out_hbm.at[c, pl.ds(off, 128)])

    return body(dst3, e016, zero16)


def _sc_edge_pass(g, src2, dst3, zeroF):
    mesh = plsc.VectorSubcoreMesh(core_axis_name="c", subcore_axis_name="s")

    @functools.partial(
        pl.kernel,
        mesh=mesh,
        compiler_params=pltpu.CompilerParams(use_tc_tiling_on_sc=False),
        out_type=jax.ShapeDtypeStruct((NC, N_ACC, FW), jnp.bfloat16),
        scratch_types=[
            pltpu.VMEM((E_PER_TILE,), jnp.int32),
            pltpu.VMEM((E_PER_TILE,), jnp.int32),
            [pltpu.VMEM((CHUNK, FW), jnp.bfloat16) for _ in range(NBUF)],
            [pltpu.SemaphoreType.DMA for _ in range(NBUF)],
            [pltpu.SemaphoreType.DMA for _ in range(NBUF)],
            pltpu.VMEM_SHARED((N_ACC, FW), jnp.bfloat16),
        ],
    )
    def body(g_hbm, src_hbm, dst_hbm, z_hbm, out_hbm,
             src_v, dst_v, rows, sem_g, sem_s, acc):
        c = lax.axis_index("c")
        s = lax.axis_index("s")
        t = c * NS + s

        pltpu.sync_copy(z_hbm, rows[0])
        base = s * ROWS_PER_TILE

        @pl.loop(0, ROWS_PER_TILE // 128)
        def _(j):
            pltpu.sync_copy(rows[0].at[pl.ds(0, 128)],
                            acc.at[pl.ds(base + j * 128, 128)])

        pltpu.sync_copy(src_hbm.at[pl.ds(t * E_PER_TILE, E_PER_TILE)], src_v)
        pltpu.sync_copy(dst_hbm.at[pl.ds(t * E_PER_TILE, E_PER_TILE)], dst_v)
        plsc.subcore_barrier()

        for b in range(AHEAD):
            pltpu.async_copy(g_hbm.at[src_v.at[pl.ds(b * CHUNK, CHUNK)]], rows[b], sem_g[b])

        @pl.loop(0, N_CHUNKS // NBUF)
        def _(o):
            for b in range(NBUF):
                i = o * NBUF + b
                j_b = (b + AHEAD) % NBUF

                def prefetch(i=i, b=b, j_b=j_b):
                    j = i + AHEAD

                    def drain():
                        pltpu.make_async_copy(
                            rows[j_b], acc.at[dst_v.at[pl.ds((j - NBUF) * CHUNK, CHUNK)]],
                            sem_s[j_b]).wait()

                    if b + AHEAD >= NBUF:
                        drain()
                    else:
                        @pl.when(o >= 1)
                        def _():
                            drain()
                    pltpu.async_copy(
                        g_hbm.at[src_v.at[pl.ds(j * CHUNK, CHUNK)]],
                        rows[j_b], sem_g[j_b])

                if b + AHEAD < NBUF:
                    prefetch()
                else:
                    @pl.when(o < N_CHUNKS // NBUF - 1)
                    def _():
                        prefetch()

                pltpu.make_async_copy(
                    g_hbm.at[src_v.at[pl.ds(i * CHUNK, CHUNK)]],
                    rows[b], sem_g[b]).wait()
                pltpu.async_copy(rows[b],
                                 acc.at[dst_v.at[pl.ds(i * CHUNK, CHUNK)]],
                                 sem_s[b], add=True)

        for k in range(NBUF):
            i = N_CHUNKS - NBUF + k
            pltpu.make_async_copy(
                rows[i % NBUF], acc.at[dst_v.at[pl.ds(i * CHUNK, CHUNK)]],
                sem_s[i % NBUF]).wait()

        plsc.subcore_barrier()

        @pl.loop(0, ROWS_PER_TILE // 128)
        def _(j):
            off = base + j * 128
            pltpu.sync_copy(acc.at[pl.ds(off, 128)], rows[0].at[pl.ds(0, 128)])
            pltpu.sync_copy(rows[0].at[pl.ds(0, 128)],
                            out_hbm.at[c, pl.ds(off, 128)])

    return body(g, src2, dst3, zeroF)


def _dot(a, b):
    return lax.dot_general(a, b, (((1,), (0,)), ((), ())),
                           precision=_HIGH, preferred_element_type=jnp.float32)


def _acc_spec():
    return pl.BlockSpec((NC, ROW_TILE, FW), lambda i: (0, i, 0))


def _full_spec():
    return pl.BlockSpec((ROW_TILE, FW), lambda i: (i, 0))


def _psum(p_ref):
    return (p_ref[0].astype(jnp.float32) + p_ref[1].astype(jnp.float32))


def _k_prep(degp, x):
    def body(degp_ref, x_ref, dinv_ref, g0_ref):
        deg = degp_ref[0, :, 0:1] + degp_ref[1, :, 0:1] + 1.0
        dinv = lax.rsqrt(deg)
        dinv_ref[...] = dinv
        g0_ref[...] = (dinv * x_ref[...]).astype(jnp.bfloat16)

    return pl.pallas_call(
        body,
        grid=(GRID,),
        in_specs=[
            pl.BlockSpec((NC, ROW_TILE, 16), lambda i: (0, i, 0)),
            pl.BlockSpec((ROW_TILE, IN_DIM), lambda i: (i, 0)),
        ],
        out_specs=[
            pl.BlockSpec((ROW_TILE, 1), lambda i: (i, 0)),
            _full_spec(),
        ],
        out_shape=[
            jax.ShapeDtypeStruct((N_NODES, 1), jnp.float32),
            jax.ShapeDtypeStruct((N_NODES, FW), jnp.bfloat16),
        ],
    )(degp, x)


def _k_layer1(p, x, dinv, W1, b1):
    def body(p_ref, x_ref, dinv_ref, W1_ref, b1_ref,
             h1_ref, g1a_ref, g1b_ref):
        dinv = dinv_ref[...]
        z = dinv * _psum(p_ref) + (dinv * dinv) * x_ref[...]
        h = jax.nn.relu(_dot(z, W1_ref[...]) + b1_ref[...])
        h1_ref[...] = h
        g1a_ref[...] = (dinv * h[:, :FW]).astype(jnp.bfloat16)
        g1b_ref[...] = (dinv * h[:, FW:]).astype(jnp.bfloat16)

    return pl.pallas_call(
        body,
        grid=(GRID,),
        in_specs=[
            _acc_spec(),
            pl.BlockSpec((ROW_TILE, IN_DIM), lambda i: (i, 0)),
            pl.BlockSpec((ROW_TILE, 1), lambda i: (i, 0)),
            pl.BlockSpec((IN_DIM, HID_DIM), lambda i: (0, 0)),
            pl.BlockSpec((1, HID_DIM), lambda i: (0, 0)),
        ],
        out_specs=[
            pl.BlockSpec((ROW_TILE, HID_DIM), lambda i: (i, 0)),
            _full_spec(),
            _full_spec(),
        ],
        out_shape=[
            jax.ShapeDtypeStruct((N_NODES, HID_DIM), jnp.float32),
            jax.ShapeDtypeStruct((N_NODES, FW), jnp.bfloat16),
            jax.ShapeDtypeStruct((N_NODES, FW), jnp.bfloat16),
        ],
    )(p, x, dinv, W1, b1)


def _k_layer23(qa, qb, h1, dinv, W2, b2, W3):
    def body(qa_ref, qb_ref, h1_ref, dinv_ref, W2_ref, b2_ref, W3_ref,
             m_ref, g2_ref):
        dinv = dinv_ref[...]
        d2 = dinv * dinv
        h1 = h1_ref[...]
        z0 = dinv * _psum(qa_ref) + d2 * h1[:, :FW]
        z1 = dinv * _psum(qb_ref) + d2 * h1[:, FW:]
        h2 = jax.nn.relu(_dot(z0, W2_ref[:FW, :]) + _dot(z1, W2_ref[FW:, :])
                         + b2_ref[...])
        m = _dot(h2, W3_ref[...])
        m_ref[...] = m
        g2_ref[...] = (dinv * m).astype(jnp.bfloat16)

    return pl.pallas_call(
        body,
        grid=(GRID,),
        in_specs=[
            _acc_spec(), _acc_spec(),
            pl.BlockSpec((ROW_TILE, HID_DIM), lambda i: (i, 0)),
            pl.BlockSpec((ROW_TILE, 1), lambda i: (i, 0)),
            pl.BlockSpec((HID_DIM, HID_DIM), lambda i: (0, 0)),
            pl.BlockSpec((1, HID_DIM), lambda i: (0, 0)),
            pl.BlockSpec((HID_DIM, OUT_DIM), lambda i: (0, 0)),
        ],
        out_specs=[
            pl.BlockSpec((ROW_TILE, OUT_DIM), lambda i: (i, 0)),
            _full_spec(),
        ],
        out_shape=[
            jax.ShapeDtypeStruct((N_NODES, OUT_DIM), jnp.float32),
            jax.ShapeDtypeStruct((N_NODES, FW), jnp.bfloat16),
        ],
    )(qa, qb, h1, dinv, W2, b2, W3)


def _k_final(r, m, dinv, b3, batch2, We, be):
    def body(r_ref, m_ref, dinv_ref, b3_ref, batch_ref,
             We_ref, be_ref, h_ref, emb_ref, pool_acc, cnt_acc):
        i = pl.program_id(0)
        dinv = dinv_ref[...]
        h3 = dinv * _psum(r_ref) + (dinv * dinv) * m_ref[...] + b3_ref[...]
        h_ref[...] = h3

        seg = lax.broadcasted_iota(jnp.int32, (ROW_TILE, NUM_GRAPHS), 1)
        oh = (batch_ref[...] == seg).astype(jnp.float32)

        @pl.when(i == 0)
        def _():
            pool_acc[...] = jnp.zeros_like(pool_acc)
            cnt_acc[...] = jnp.zeros_like(cnt_acc)

        contract = (((0,), (0,)), ((), ()))
        pool_acc[...] += lax.dot_general(
            oh, h3, contract, precision=_HIGH,
            preferred_element_type=jnp.float32)
        cnt_acc[...] += lax.dot_general(
            oh, jnp.ones((ROW_TILE, OUT_DIM), jnp.float32), contract,
            precision=_HIGH, preferred_element_type=jnp.float32)

        @pl.when(i == GRID - 1)
        def _():
            mean = pool_acc[...] / jnp.maximum(cnt_acc[...], 1.0)
            emb_ref[...] = _dot(mean, We_ref[...]) + be_ref[...]

    return pl.pallas_call(
        body,
        grid=(GRID,),
        in_specs=[
            _acc_spec(),
            pl.BlockSpec((ROW_TILE, OUT_DIM), lambda i: (i, 0)),
            pl.BlockSpec((ROW_TILE, 1), lambda i: (i, 0)),
            pl.BlockSpec((1, OUT_DIM), lambda i: (0, 0)),
            pl.BlockSpec((ROW_TILE, 1), lambda i: (i, 0)),
            pl.BlockSpec((OUT_DIM, OUT_DIM), lambda i: (0, 0)),
            pl.BlockSpec((1, OUT_DIM), lambda i: (0, 0)),
        ],
        out_specs=[
            pl.BlockSpec((ROW_TILE, OUT_DIM), lambda i: (i, 0)),
            pl.BlockSpec((NUM_GRAPHS, OUT_DIM), lambda i: (0, 0)),
        ],
        out_shape=[
            jax.ShapeDtypeStruct((N_NODES, OUT_DIM), jnp.float32),
            jax.ShapeDtypeStruct((NUM_GRAPHS, OUT_DIM), jnp.float32),
        ],
        scratch_shapes=[
            pltpu.VMEM((NUM_GRAPHS, OUT_DIM), jnp.float32),
            pltpu.VMEM((NUM_GRAPHS, OUT_DIM), jnp.float32),
        ],
    )(r, m, dinv, b3, batch2, We, be)


def kernel(x, edge_index, batch, W1, b1, W2, b2, W3, b3, We, be):
    src = edge_index[0].astype(jnp.int32)
    dst = edge_index[1].astype(jnp.int32)
    npad = E_PAD - N_EDGES
    src2 = jnp.concatenate([src, jnp.zeros((npad,), jnp.int32)])
    dst3 = jnp.concatenate(
        [dst, jnp.full((npad,), N_NODES, jnp.int32)])
    batch2 = batch.astype(jnp.int32).reshape(N_NODES, 1)
    b1r = b1.reshape(1, HID_DIM)
    b2r = b2.reshape(1, HID_DIM)
    b3r = b3.reshape(1, OUT_DIM)
    ber = be.reshape(1, OUT_DIM)
    e016 = jnp.tile((jnp.arange(16) == 0).astype(jnp.float32), (CHUNK, 1))
    zero16 = jnp.zeros((CHUNK, 16), jnp.float32)
    zeroF = jnp.zeros((CHUNK, FW), jnp.bfloat16)

    degp = _sc_degree(dst3, e016, zero16)
    dinv, g0 = _k_prep(degp, x)
    p = _sc_edge_pass(g0, src2, dst3, zeroF)
    h1, g1a, g1b = _k_layer1(p, x, dinv, W1, b1r)
    qa = _sc_edge_pass(g1a, src2, dst3, zeroF)
    qb = _sc_edge_pass(g1b, src2, dst3, zeroF)
    m, g2 = _k_layer23(qa, qb, h1, dinv, W2, b2r, W3)
    r = _sc_edge_pass(g2, src2, dst3, zeroF)
    h, emb = _k_final(r, m, dinv, b3r, batch2, We, ber)
    return (emb, h)
